# baseline (speedup 1.0000x reference)
import jax
import jax.numpy as jnp
from jax.experimental import pallas as pl
from jax.experimental.pallas import tpu as pltpu

_D = 1024
_B = 4
_LN_EPS = 1e-5
_TS = 512
_CH = 128


def _ln_kernel(x_ref, pe_ref, g_ref, b_ref, o_ref):
    g = g_ref[...]
    b = b_ref[...]
    for c in range(_TS // _CH):
        sl = pl.ds(c * _CH, _CH)
        pe = pe_ref[sl, :]
        h = x_ref[sl, :, :] + pe[:, None, :]
        mean = jnp.mean(h, axis=-1, keepdims=True)
        hc = h - mean
        var = jnp.mean(hc * hc, axis=-1, keepdims=True)
        o_ref[sl, :, :] = hc * jax.lax.rsqrt(var + _LN_EPS) * g[None] + b[None]


def kernel(x, pos_table, ln_gamma, ln_beta):
    S, B, D = x.shape
    g2 = ln_gamma.reshape(1, D)
    b2 = ln_beta.reshape(1, D)
    out = pl.pallas_call(
        _ln_kernel,
        grid=(S // _TS,),
        in_specs=[
            pl.BlockSpec((_TS, B, D), lambda s: (s, 0, 0)),
            pl.BlockSpec((_TS, D), lambda s: (s, 0)),
            pl.BlockSpec((1, D), lambda s: (0, 0)),
            pl.BlockSpec((1, D), lambda s: (0, 0)),
        ],
        out_specs=pl.BlockSpec((_TS, B, D), lambda s: (s, 0, 0)),
        out_shape=jax.ShapeDtypeStruct((S, B, D), x.dtype),
        compiler_params=pltpu.CompilerParams(
            dimension_semantics=("parallel",)),
    )(x, pos_table, g2, b2)
    return out


# speedup vs baseline: 1.0045x; 1.0045x over previous
import jax
import jax.numpy as jnp
from jax.experimental import pallas as pl
from jax.experimental.pallas import tpu as pltpu

_D = 1024
_B = 4
_LN_EPS = 1e-5
_TS = 512
_CH = 32


def _ln_kernel(x_ref, pe_ref, g_ref, b_ref, o_ref):
    g = g_ref[...]
    b = b_ref[...]
    for c in range(_TS // _CH):
        sl = pl.ds(c * _CH, _CH)
        pe = pe_ref[sl, :]
        h = x_ref[sl, :, :] + pe[:, None, :]
        mean = jnp.mean(h, axis=-1, keepdims=True)
        hc = h - mean
        var = jnp.mean(hc * hc, axis=-1, keepdims=True)
        o_ref[sl, :, :] = hc * jax.lax.rsqrt(var + _LN_EPS) * g[None] + b[None]


def kernel(x, pos_table, ln_gamma, ln_beta):
    S, B, D = x.shape
    g2 = ln_gamma.reshape(1, D)
    b2 = ln_beta.reshape(1, D)
    out = pl.pallas_call(
        _ln_kernel,
        grid=(S // _TS,),
        in_specs=[
            pl.BlockSpec((_TS, B, D), lambda s: (s, 0, 0)),
            pl.BlockSpec((_TS, D), lambda s: (s, 0)),
            pl.BlockSpec((1, D), lambda s: (0, 0)),
            pl.BlockSpec((1, D), lambda s: (0, 0)),
        ],
        out_specs=pl.BlockSpec((_TS, B, D), lambda s: (s, 0, 0)),
        out_shape=jax.ShapeDtypeStruct((S, B, D), x.dtype),
        compiler_params=pltpu.CompilerParams(
            dimension_semantics=("parallel",)),
    )(x, pos_table, g2, b2)
    return out
